# Initial kernel scaffold; baseline (speedup 1.0000x reference)
import jax
import jax.numpy as jnp
from jax.experimental import pallas as pl


def kernel(feat, edge_index, edge_type, emb, Wq, Wk, Wv, Wa, ba, Wp, bp):
    raise NotImplementedError("write your pallas kernel here")


# speedup vs baseline: 2.9762x; 2.9762x over previous
import functools

import jax
import jax.numpy as jnp
from jax import lax
from jax.experimental import pallas as pl
from jax.experimental.pallas import tpu as pltpu
from jax.experimental.pallas import tpu_sc as plsc

_NC = 2
_NS = 16
_L = 16


def _sc_segsum(feat_ext, src, seg, n_seg):
    n_rows, W = feat_ext.shape
    E = src.shape[0]
    CH = 4096
    NCHUNK = -(-n_seg // CH)
    NCHUNK = -(-NCHUNK // _NC) * _NC
    PASSES = NCHUNK // _NC
    GOUT = NCHUNK * CH
    EPC = E // _NS
    BE = 2000
    NB = EPC // BE
    NV = BE // _L
    GB = 128
    NJ = GB // 128
    STAGE = GB + _L
    RPS = CH // _NS
    DUMMY = CH

    mesh = plsc.VectorSubcoreMesh(core_axis_name="c", subcore_axis_name="s")

    NROW = STAGE // 128 + 1

    @functools.partial(
        pl.kernel,
        out_type=jax.ShapeDtypeStruct((GOUT, W), jnp.float32),
        mesh=mesh,
        compiler_params=pltpu.CompilerParams(
            needs_layout_passes=False, use_tc_tiling_on_sc=False),
        scratch_types=[
            pltpu.VMEM((BE,), jnp.int32),
            pltpu.VMEM((BE,), jnp.int32),
            pltpu.VMEM((STAGE,), jnp.int32),
            pltpu.VMEM((NROW, 128), jnp.int32),
            pltpu.VMEM((GB, W), jnp.float32),
            pltpu.VMEM((_L, W), jnp.float32),
            pltpu.VMEM_SHARED((CH + _L, W), jnp.float32),
            pltpu.SemaphoreType.DMA,
            pltpu.SemaphoreType.DMA,
        ],
    )
    def sc_fn(feat_hbm, src_hbm, seg_hbm, out_hbm,
              meta_src, meta_seg, stage_src, stage_seg, rows, zblk,
              acc, sem0, sem1):
        c = lax.axis_index("c")
        s = lax.axis_index("s")
        sems = [sem0, sem1]

        zv = jnp.zeros((_L,), jnp.float32)
        for i in range(_L):
            for j in range(W // _L):
                zblk[i, _L * j:_L * (j + 1)] = zv

        def flush():
            cps = [
                pltpu.async_copy(
                    feat_hbm.at[stage_src.at[pl.ds(128 * j, 128)]],
                    rows.at[pl.ds(128 * j, 128)], sems[j])
                for j in range(NJ)
            ]
            for cp in cps:
                cp.wait()
            for j in range(NJ):
                pltpu.sync_copy(rows.at[pl.ds(128 * j, 128)],
                                acc.at[stage_seg.at[j]], add=True)

        def flush_and_tail(off):
            flush()
            ts = stage_src[pl.ds(GB, _L)]
            tg = stage_seg[NJ, 0:_L]
            stage_src[pl.ds(0, _L)] = ts
            stage_seg[0, 0:_L] = tg
            return off - GB

        for p in range(PASSES):
            chunk = c * PASSES + p
            lo = chunk * CH
            for t in range(RPS // _L):
                pltpu.sync_copy(zblk, acc.at[pl.ds(s * RPS + _L * t, _L)])
            plsc.subcore_barrier()

            def step(i, off):
                s16 = meta_src[pl.ds(_L * i, _L)]
                g16 = meta_seg[pl.ds(_L * i, _L)]
                gl = g16 - lo
                msk = (gl >= 0) & (gl < CH)
                mi = msk.astype(jnp.int32)
                incl = plsc.cumsum(mi)
                dst = off + incl - mi
                plsc.store_scatter(stage_src, [dst], s16, mask=msk)
                plsc.store_scatter(stage_seg, [dst >> 7, dst & 127], gl,
                                   mask=msk)
                off = off + jnp.max(incl)
                off = lax.cond(off >= GB, flush_and_tail, lambda o: o, off)
                return off

            off = jnp.int32(0)
            for b in range(NB):
                base = s * EPC + b * BE
                pltpu.sync_copy(src_hbm.at[pl.ds(base, BE)], meta_src)
                pltpu.sync_copy(seg_hbm.at[pl.ds(base, BE)], meta_seg)
                off = lax.fori_loop(0, NV, step, off)

            for bb in range(GB // _L):
                lane = lax.iota(jnp.int32, _L) + _L * bb
                m = lane < off
                row, col = (_L * bb) // 128, (_L * bb) % 128
                sv = stage_src[pl.ds(_L * bb, _L)]
                gv = stage_seg[row, col:col + _L]
                stage_src[pl.ds(_L * bb, _L)] = jnp.where(m, sv, 0)
                stage_seg[row, col:col + _L] = jnp.where(m, gv, DUMMY)
            flush()
            plsc.subcore_barrier()
            pltpu.sync_copy(acc.at[pl.ds(s * RPS, RPS)],
                            out_hbm.at[pl.ds(lo + s * RPS, RPS)])

    return sc_fn(feat_ext, src, seg)


def _tc_attn(feat, G3, emb, Wq, Wk, Wv, Wa, ba2, Wp, bp2):
    N, D = feat.shape
    R = emb.shape[0]
    W = G3.shape[2]
    BN = 2000
    grid = N // BN

    def body(feat_ref, g_ref, emb_ref, wq, wk, wv, wa, ba_ref, wp, bp_ref,
             out_ref):
        f = feat_ref[...]
        dn = (((1,), (0,)), ((), ()))
        dt = (((1,), (1,)), ((), ()))
        waq = lax.dot_general(wa[...], wq[...], dn,
                              preferred_element_type=jnp.float32)
        wak = lax.dot_general(wa[...], wk[...], dn,
                              preferred_element_type=jnp.float32)
        qa = lax.dot_general(f, waq, dt, preferred_element_type=jnp.float32)
        ss = []
        vs = []
        for r in range(R):
            g = g_ref[r]
            sums = g[:, :D]
            cnt = g[:, D:D + 1]
            km = (sums / jnp.maximum(cnt, 1.0)) * emb_ref[r][None, :]
            s_r = qa - lax.dot_general(km, wak, dt,
                                       preferred_element_type=jnp.float32)
            ss.append(s_r + ba_ref[...])
            vs.append(lax.dot_general(km, wv[...], dt,
                                      preferred_element_type=jnp.float32))
        m = jnp.maximum(jnp.maximum(ss[0], ss[1]), ss[2])
        es = [jnp.exp(s_r - m) for s_r in ss]
        den = es[0] + es[1] + es[2]
        o = (es[0] * vs[0] + es[1] * vs[1] + es[2] * vs[2]) / den
        out_ref[...] = (
            lax.dot_general(o, wp[...], dt, preferred_element_type=jnp.float32)
            + bp_ref[...] + f)

    wspec = pl.BlockSpec((D, D), lambda i: (0, 0))
    bspec = pl.BlockSpec((1, D), lambda i: (0, 0))
    return pl.pallas_call(
        body,
        grid=(grid,),
        in_specs=[
            pl.BlockSpec((BN, D), lambda i: (i, 0)),
            pl.BlockSpec((R, BN, W), lambda i: (0, i, 0)),
            pl.BlockSpec((R, D), lambda i: (0, 0)),
            wspec, wspec, wspec, wspec, bspec, wspec, bspec,
        ],
        out_specs=pl.BlockSpec((BN, D), lambda i: (i, 0)),
        out_shape=jax.ShapeDtypeStruct((N, D), jnp.float32),
    )(feat, G3, emb, Wq, Wk, Wv, Wa, ba2, Wp, bp2)


def kernel(feat, edge_index, edge_type, emb, Wq, Wk, Wv, Wa, ba, Wp, bp):
    N, D = feat.shape
    R = emb.shape[0]
    W = D + _L
    src = edge_index[0]
    dst = edge_index[1]
    seg = edge_type * N + dst
    feat_ext = jnp.concatenate(
        [feat, jnp.ones((N, 1), jnp.float32), jnp.zeros((N, _L - 1),
                                                        jnp.float32)],
        axis=1)
    G = _sc_segsum(feat_ext, src, seg, R * N)
    G3 = G[:R * N].reshape(R, N, W)
    return _tc_attn(feat, G3, emb, Wq, Wk, Wv, Wa,
                    ba.reshape(1, -1), Wp, bp.reshape(1, -1))


# speedup vs baseline: 3.4957x; 1.1745x over previous
import functools

import jax
import jax.numpy as jnp
from jax import lax
from jax.experimental import pallas as pl
from jax.experimental.pallas import tpu as pltpu
from jax.experimental.pallas import tpu_sc as plsc

_NC = 2
_NS = 16
_L = 16


def _sc_segsum(feat_ext, src, seg, n_seg):
    n_rows, W = feat_ext.shape
    E = src.shape[0]
    CH = 5120
    NCHUNK = -(-n_seg // CH)
    NCHUNK = -(-NCHUNK // _NC) * _NC
    PASSES = NCHUNK // _NC
    GOUT = NCHUNK * CH
    EPC = E // _NS
    BE = 2000
    NB = EPC // BE
    NV = BE // _L
    GB = 128
    NJ = GB // 128
    STAGE = GB + _L
    RPS = CH // _NS
    DUMMY = CH

    mesh = plsc.VectorSubcoreMesh(core_axis_name="c", subcore_axis_name="s")

    NROW = STAGE // 128 + 1

    @functools.partial(
        pl.kernel,
        out_type=jax.ShapeDtypeStruct((GOUT, W), jnp.float32),
        mesh=mesh,
        compiler_params=pltpu.CompilerParams(
            needs_layout_passes=False, use_tc_tiling_on_sc=False),
        scratch_types=[
            pltpu.VMEM((BE,), jnp.int32),
            pltpu.VMEM((BE,), jnp.int32),
            pltpu.VMEM((STAGE,), jnp.int32),
            pltpu.VMEM((NROW, 128), jnp.int32),
            pltpu.VMEM((GB, W), jnp.float32),
            pltpu.VMEM((8, W), jnp.float32),
            pltpu.VMEM_SHARED((CH + _L, W), jnp.float32),
            pltpu.SemaphoreType.DMA,
            pltpu.SemaphoreType.DMA,
        ],
    )
    def sc_fn(feat_hbm, src_hbm, seg_hbm, out_hbm,
              meta_src, meta_seg, stage_src, stage_seg, rows, zblk,
              acc, sem0, sem1):
        c = lax.axis_index("c")
        s = lax.axis_index("s")
        sems = [sem0, sem1]

        zv = jnp.zeros((_L,), jnp.float32)
        for i in range(8):
            for j in range(W // _L):
                zblk[i, _L * j:_L * (j + 1)] = zv

        def flush():
            cps = [
                pltpu.async_copy(
                    feat_hbm.at[stage_src.at[pl.ds(128 * j, 128)]],
                    rows.at[pl.ds(128 * j, 128)], sems[j])
                for j in range(NJ)
            ]
            for cp in cps:
                cp.wait()
            for j in range(NJ):
                pltpu.sync_copy(rows.at[pl.ds(128 * j, 128)],
                                acc.at[stage_seg.at[j]], add=True)

        def flush_and_tail(off):
            flush()
            ts = stage_src[pl.ds(GB, _L)]
            tg = stage_seg[NJ, 0:_L]
            stage_src[pl.ds(0, _L)] = ts
            stage_seg[0, 0:_L] = tg
            return off - GB

        for p in range(PASSES):
            chunk = c * PASSES + p
            lo = chunk * CH
            for t in range(RPS // 8):
                pltpu.sync_copy(zblk, acc.at[pl.ds(s * RPS + 8 * t, 8)])
            plsc.subcore_barrier()

            def step(i, off):
                s16 = meta_src[pl.ds(_L * i, _L)]
                g16 = meta_seg[pl.ds(_L * i, _L)]
                gl = g16 - lo
                msk = (gl >= 0) & (gl < CH)
                mi = msk.astype(jnp.int32)
                incl = plsc.cumsum(mi)
                dst = off + incl - mi
                plsc.store_scatter(stage_src, [dst], s16, mask=msk)
                plsc.store_scatter(stage_seg, [dst >> 7, dst & 127], gl,
                                   mask=msk)
                off = off + incl[_L - 1]
                off = lax.cond(off >= GB, flush_and_tail, lambda o: o, off)
                return off

            off = jnp.int32(0)
            for b in range(NB):
                base = s * EPC + b * BE
                pltpu.sync_copy(src_hbm.at[pl.ds(base, BE)], meta_src)
                pltpu.sync_copy(seg_hbm.at[pl.ds(base, BE)], meta_seg)
                off = lax.fori_loop(0, NV, step, off)

            for bb in range(GB // _L):
                lane = lax.iota(jnp.int32, _L) + _L * bb
                m = lane < off
                row, col = (_L * bb) // 128, (_L * bb) % 128
                sv = stage_src[pl.ds(_L * bb, _L)]
                gv = stage_seg[row, col:col + _L]
                stage_src[pl.ds(_L * bb, _L)] = jnp.where(m, sv, 0)
                stage_seg[row, col:col + _L] = jnp.where(m, gv, DUMMY)
            flush()
            plsc.subcore_barrier()
            pltpu.sync_copy(acc.at[pl.ds(s * RPS, RPS)],
                            out_hbm.at[pl.ds(lo + s * RPS, RPS)])

    return sc_fn(feat_ext, src, seg)


def _tc_attn(feat, G, emb, Wq, Wk, Wv, Wa, ba2, Wp, bp2):
    N, D = feat.shape
    R = emb.shape[0]
    W = G.shape[1]
    BN = 2000
    grid = N // BN
    nb = N // BN

    def body(feat_ref, g0, g1, g2, emb_ref, wq, wk, wv, wa, ba_ref, wp,
             bp_ref, out_ref):
        f = feat_ref[...]
        dn = (((1,), (0,)), ((), ()))
        dt = (((1,), (1,)), ((), ()))
        waq = lax.dot_general(wa[...], wq[...], dn,
                              preferred_element_type=jnp.float32)
        wak = lax.dot_general(wa[...], wk[...], dn,
                              preferred_element_type=jnp.float32)
        qa = lax.dot_general(f, waq, dt, preferred_element_type=jnp.float32)
        ss = []
        vs = []
        for r, g_ref in enumerate((g0, g1, g2)):
            g = g_ref[...]
            sums = g[:, :D]
            cnt = g[:, D:D + 1]
            km = (sums / jnp.maximum(cnt, 1.0)) * emb_ref[r][None, :]
            s_r = qa - lax.dot_general(km, wak, dt,
                                       preferred_element_type=jnp.float32)
            ss.append(s_r + ba_ref[...])
            vs.append(lax.dot_general(km, wv[...], dt,
                                      preferred_element_type=jnp.float32))
        m = jnp.maximum(jnp.maximum(ss[0], ss[1]), ss[2])
        es = [jnp.exp(s_r - m) for s_r in ss]
        den = es[0] + es[1] + es[2]
        o = (es[0] * vs[0] + es[1] * vs[1] + es[2] * vs[2]) / den
        out_ref[...] = (
            lax.dot_general(o, wp[...], dt, preferred_element_type=jnp.float32)
            + bp_ref[...] + f)

    def gview(r):
        return pl.BlockSpec((BN, W), lambda i, r=r: (nb * r + i, 0))

    wspec = pl.BlockSpec((D, D), lambda i: (0, 0))
    bspec = pl.BlockSpec((1, D), lambda i: (0, 0))
    return pl.pallas_call(
        body,
        grid=(grid,),
        in_specs=[
            pl.BlockSpec((BN, D), lambda i: (i, 0)),
            gview(0), gview(1), gview(2),
            pl.BlockSpec((R, D), lambda i: (0, 0)),
            wspec, wspec, wspec, wspec, bspec, wspec, bspec,
        ],
        out_specs=pl.BlockSpec((BN, D), lambda i: (i, 0)),
        out_shape=jax.ShapeDtypeStruct((N, D), jnp.float32),
    )(feat, G, G, G, emb, Wq, Wk, Wv, Wa, ba2, Wp, bp2)


def kernel(feat, edge_index, edge_type, emb, Wq, Wk, Wv, Wa, ba, Wp, bp):
    N, D = feat.shape
    R = emb.shape[0]
    W = D + _L
    src = edge_index[0]
    dst = edge_index[1]
    seg = edge_type * N + dst
    feat_ext = jnp.concatenate(
        [feat, jnp.ones((N, 1), jnp.float32), jnp.zeros((N, _L - 1),
                                                        jnp.float32)],
        axis=1)
    G = _sc_segsum(feat_ext, src, seg, R * N)
    return _tc_attn(feat, G, emb, Wq, Wk, Wv, Wa,
                    ba.reshape(1, -1), Wp, bp.reshape(1, -1))


# speedup vs baseline: 4.5111x; 1.2905x over previous
import functools

import jax
import jax.numpy as jnp
from jax import lax
from jax.experimental import pallas as pl
from jax.experimental.pallas import tpu as pltpu
from jax.experimental.pallas import tpu_sc as plsc

_NC = 2
_NS = 16
_L = 16


def _sc_segsum(feat_ext, src, seg, n_seg):
    n_rows, W = feat_ext.shape
    E = src.shape[0]
    CH = 4096
    NCHUNK = -(-n_seg // CH)
    NCHUNK = -(-NCHUNK // _NC) * _NC
    PASSES = NCHUNK // _NC
    GOUT = NCHUNK * CH
    EPC = E // _NS
    BE = 400
    NB = EPC // BE
    NV = BE // _L
    GB = 64
    SCAP = EPC + 2 * GB
    RPS = CH // _NS
    DUMMY = CH

    mesh = plsc.VectorSubcoreMesh(core_axis_name="c", subcore_axis_name="s")

    @functools.partial(
        pl.kernel,
        out_type=jax.ShapeDtypeStruct((GOUT, W), jnp.float32),
        mesh=mesh,
        compiler_params=pltpu.CompilerParams(
            needs_layout_passes=False, use_tc_tiling_on_sc=False),
        scratch_types=[
            pltpu.VMEM((2, BE), jnp.int32),
            pltpu.VMEM((2, BE), jnp.int32),
            pltpu.VMEM((SCAP,), jnp.int32),
            pltpu.VMEM((SCAP,), jnp.int32),
            pltpu.VMEM((2 * GB, W), jnp.float32),
            pltpu.VMEM((8, W), jnp.float32),
            pltpu.VMEM_SHARED((CH + _L, W), jnp.float32),
            pltpu.SemaphoreType.DMA,
            pltpu.SemaphoreType.DMA,
            pltpu.SemaphoreType.DMA,
            pltpu.SemaphoreType.DMA,
            pltpu.SemaphoreType.DMA,
            pltpu.SemaphoreType.DMA,
        ],
    )
    def sc_fn(feat_hbm, src_hbm, seg_hbm, out_hbm,
              meta_src, meta_seg, stage_src, stage_seg, rows, zblk, acc,
              gsem0, gsem1, ssem0, ssem1, msem, zsem):
        c = lax.axis_index("c")
        s = lax.axis_index("s")
        gsems = [gsem0, gsem1]
        ssems = [ssem0, ssem1]

        zv = jnp.zeros((_L,), jnp.float32)
        for i in range(8):
            for j in range(W // _L):
                zblk[i, _L * j:_L * (j + 1)] = zv

        def issue_meta(b, buf):
            base = s * EPC + b * BE
            c1 = pltpu.async_copy(src_hbm.at[pl.ds(base, BE)],
                                  meta_src.at[buf], msem)
            c2 = pltpu.async_copy(seg_hbm.at[pl.ds(base, BE)],
                                  meta_seg.at[buf], msem)
            return (c1, c2)

        for p in range(PASSES):
            chunk = c * PASSES + p
            lo = chunk * CH

            pend = issue_meta(0, 0)
            zcps = [
                pltpu.async_copy(zblk, acc.at[pl.ds(s * RPS + 8 * t, 8)],
                                 zsem)
                for t in range(RPS // 8)
            ]
            for cp in zcps:
                cp.wait()
            plsc.subcore_barrier()

            def make_step(buf):
                def stepf(i, off):
                    s16 = meta_src[buf, pl.ds(_L * i, _L)]
                    g16 = meta_seg[buf, pl.ds(_L * i, _L)]
                    gl = g16 - lo
                    msk = (gl >= 0) & (gl < CH)
                    mi = msk.astype(jnp.int32)
                    incl = plsc.cumsum(mi)
                    dst = off + incl - mi
                    plsc.store_scatter(stage_src, [dst], s16, mask=msk)
                    plsc.store_scatter(stage_seg, [dst], gl, mask=msk)
                    return off + incl[_L - 1]
                return stepf

            off = jnp.int32(0)
            for b in range(NB):
                buf = b & 1
                for cp in pend:
                    cp.wait()
                if b + 1 < NB:
                    pend = issue_meta(b + 1, 1 - buf)
                off = lax.fori_loop(0, NV, make_step(buf), off)

            base0 = off - (off & (GB - 1))
            for kk in range(GB // _L):
                base = base0 + _L * kk
                lane = base + lax.iota(jnp.int32, _L)
                m = lane < off
                sv = stage_src[pl.ds(base, _L)]
                gv = stage_seg[pl.ds(base, _L)]
                stage_src[pl.ds(base, _L)] = jnp.where(m, sv, 0)
                stage_seg[pl.ds(base, _L)] = jnp.where(m, gv, DUMMY)
            nblk = (off + GB - 1) >> 6

            def fbody(j, _):
                for sl in (0, 1):
                    pn = 1 - sl

                    @pl.when((j & 1) == sl)
                    def _():
                        @pl.when((j >= 2) & (j - 2 < nblk))
                        def _():
                            pltpu.make_async_copy(
                                feat_hbm.at[pl.ds(0, GB)],
                                rows.at[pl.ds(GB * sl, GB)],
                                ssems[sl]).wait()

                        @pl.when(j < nblk)
                        def _():
                            pltpu.async_copy(
                                feat_hbm.at[stage_src.at[pl.ds(GB * j, GB)]],
                                rows.at[pl.ds(GB * sl, GB)], gsems[sl])

                        @pl.when((j >= 1) & (j <= nblk))
                        def _():
                            pltpu.make_async_copy(
                                feat_hbm.at[pl.ds(0, GB)],
                                rows.at[pl.ds(GB * pn, GB)],
                                gsems[pn]).wait()
                            for k in range(GB // _L):
                                idx16 = stage_seg[
                                    pl.ds(GB * (j - 1) + _L * k, _L)]
                                pltpu.async_copy(
                                    rows.at[pl.ds(GB * pn + _L * k, _L)],
                                    acc.at[idx16], ssems[pn], add=True)
                return 0

            lax.fori_loop(0, nblk + 2, fbody, 0)
            plsc.subcore_barrier()

            pltpu.sync_copy(acc.at[pl.ds(s * RPS, RPS)],
                            out_hbm.at[pl.ds(lo + s * RPS, RPS)])

    return sc_fn(feat_ext, src, seg)


def _tc_attn(feat, G, emb, Wq, Wk, Wv, Wa, ba2, Wp, bp2):
    N, D = feat.shape
    R = emb.shape[0]
    W = G.shape[1]
    BN = 2000
    grid = N // BN
    nb = N // BN

    def body(feat_ref, g0, g1, g2, emb_ref, wq, wk, wv, wa, ba_ref, wp,
             bp_ref, out_ref):
        f = feat_ref[...]
        dn = (((1,), (0,)), ((), ()))
        dt = (((1,), (1,)), ((), ()))
        waq = lax.dot_general(wa[...], wq[...], dn,
                              preferred_element_type=jnp.float32)
        wak = lax.dot_general(wa[...], wk[...], dn,
                              preferred_element_type=jnp.float32)
        qa = lax.dot_general(f, waq, dt, preferred_element_type=jnp.float32)
        ss = []
        vs = []
        for r, g_ref in enumerate((g0, g1, g2)):
            g = g_ref[...]
            sums = g[:, :D]
            cnt = g[:, D:D + 1]
            km = (sums / jnp.maximum(cnt, 1.0)) * emb_ref[r][None, :]
            s_r = qa - lax.dot_general(km, wak, dt,
                                       preferred_element_type=jnp.float32)
            ss.append(s_r + ba_ref[...])
            vs.append(lax.dot_general(km, wv[...], dt,
                                      preferred_element_type=jnp.float32))
        m = jnp.maximum(jnp.maximum(ss[0], ss[1]), ss[2])
        es = [jnp.exp(s_r - m) for s_r in ss]
        den = es[0] + es[1] + es[2]
        o = (es[0] * vs[0] + es[1] * vs[1] + es[2] * vs[2]) / den
        out_ref[...] = (
            lax.dot_general(o, wp[...], dt, preferred_element_type=jnp.float32)
            + bp_ref[...] + f)

    def gview(r):
        return pl.BlockSpec((BN, W), lambda i, r=r: (nb * r + i, 0))

    wspec = pl.BlockSpec((D, D), lambda i: (0, 0))
    bspec = pl.BlockSpec((1, D), lambda i: (0, 0))
    return pl.pallas_call(
        body,
        grid=(grid,),
        in_specs=[
            pl.BlockSpec((BN, D), lambda i: (i, 0)),
            gview(0), gview(1), gview(2),
            pl.BlockSpec((R, D), lambda i: (0, 0)),
            wspec, wspec, wspec, wspec, bspec, wspec, bspec,
        ],
        out_specs=pl.BlockSpec((BN, D), lambda i: (i, 0)),
        out_shape=jax.ShapeDtypeStruct((N, D), jnp.float32),
    )(feat, G, G, G, emb, Wq, Wk, Wv, Wa, ba2, Wp, bp2)


def kernel(feat, edge_index, edge_type, emb, Wq, Wk, Wv, Wa, ba, Wp, bp):
    N, D = feat.shape
    R = emb.shape[0]
    W = D + _L
    src = edge_index[0]
    dst = edge_index[1]
    seg = edge_type * N + dst
    feat_ext = jnp.concatenate(
        [feat, jnp.ones((N, 1), jnp.float32), jnp.zeros((N, _L - 1),
                                                        jnp.float32)],
        axis=1)
    G = _sc_segsum(feat_ext, src, seg, R * N)
    return _tc_attn(feat, G, emb, Wq, Wk, Wv, Wa,
                    ba.reshape(1, -1), Wp, bp.reshape(1, -1))


# speedup vs baseline: 5.2872x; 1.1720x over previous
import functools

import jax
import jax.numpy as jnp
from jax import lax
from jax.experimental import pallas as pl
from jax.experimental.pallas import tpu as pltpu
from jax.experimental.pallas import tpu_sc as plsc

_NC = 2
_NS = 16
_L = 16


def _sc_segsum(feat_ext, src, seg, n_seg):
    n_rows, W = feat_ext.shape
    E = src.shape[0]
    CH = 4096
    NCHUNK = -(-n_seg // CH)
    NCHUNK = -(-NCHUNK // _NC) * _NC
    PASSES = NCHUNK // _NC
    GOUT = NCHUNK * CH
    EPC = E // _NS
    BE = 400
    NB = EPC // BE
    NV = BE // _L
    GB = 32
    GSH = GB.bit_length() - 1
    NSL = 4
    SCAP = EPC + 2 * GB
    RPS = CH // _NS
    DUMMY = CH

    mesh = plsc.VectorSubcoreMesh(core_axis_name="c", subcore_axis_name="s")

    @functools.partial(
        pl.kernel,
        out_type=jax.ShapeDtypeStruct((GOUT, W), jnp.float32),
        mesh=mesh,
        compiler_params=pltpu.CompilerParams(
            needs_layout_passes=False, use_tc_tiling_on_sc=False),
        scratch_types=[
            pltpu.VMEM((2, BE), jnp.int32),
            pltpu.VMEM((2, BE), jnp.int32),
            pltpu.VMEM((SCAP,), jnp.int32),
            pltpu.VMEM((SCAP,), jnp.int32),
            pltpu.VMEM((NSL * GB, W), jnp.float32),
            pltpu.VMEM((8, W), jnp.float32),
            pltpu.VMEM_SHARED((CH + _L, W), jnp.float32),
        ] + [pltpu.SemaphoreType.DMA] * (2 * NSL + 2),
    )
    def sc_fn(feat_hbm, src_hbm, seg_hbm, out_hbm,
              meta_src, meta_seg, stage_src, stage_seg, rows, zblk, acc,
              *sems):
        c = lax.axis_index("c")
        s = lax.axis_index("s")
        gsems = sems[:NSL]
        ssems = sems[NSL:2 * NSL]
        msem = sems[2 * NSL]
        zsem = sems[2 * NSL + 1]

        zv = jnp.zeros((_L,), jnp.float32)
        for i in range(8):
            for j in range(W // _L):
                zblk[i, _L * j:_L * (j + 1)] = zv

        def issue_meta(b, buf):
            base = s * EPC + b * BE
            c1 = pltpu.async_copy(src_hbm.at[pl.ds(base, BE)],
                                  meta_src.at[buf], msem)
            c2 = pltpu.async_copy(seg_hbm.at[pl.ds(base, BE)],
                                  meta_seg.at[buf], msem)
            return (c1, c2)

        for p in range(PASSES):
            chunk = c * PASSES + p
            lo = chunk * CH

            pend = issue_meta(0, 0)
            zcps = [
                pltpu.async_copy(zblk, acc.at[pl.ds(s * RPS + 8 * t, 8)],
                                 zsem)
                for t in range(RPS // 8)
            ]

            def make_step(buf):
                def stepf(i, off):
                    s16 = meta_src[buf, pl.ds(_L * i, _L)]
                    g16 = meta_seg[buf, pl.ds(_L * i, _L)]
                    gl = g16 - lo
                    msk = (gl >= 0) & (gl < CH)
                    mi = msk.astype(jnp.int32)
                    incl = plsc.cumsum(mi)
                    dst = off + incl - mi
                    plsc.store_scatter(stage_src, [dst], s16, mask=msk)
                    plsc.store_scatter(stage_seg, [dst], gl, mask=msk)
                    return off + incl[_L - 1]
                return stepf

            off = jnp.int32(0)
            for b in range(NB):
                buf = b & 1
                for cp in pend:
                    cp.wait()
                if b + 1 < NB:
                    pend = issue_meta(b + 1, 1 - buf)
                off = lax.fori_loop(0, NV, make_step(buf), off)

            base0 = off - (off & (GB - 1))
            for kk in range(GB // _L):
                base = base0 + _L * kk
                lane = base + lax.iota(jnp.int32, _L)
                m = lane < off
                sv = stage_src[pl.ds(base, _L)]
                gv = stage_seg[pl.ds(base, _L)]
                stage_src[pl.ds(base, _L)] = jnp.where(m, sv, 0)
                stage_seg[pl.ds(base, _L)] = jnp.where(m, gv, DUMMY)
            nblk = (off + GB - 1) >> GSH

            for cp in zcps:
                cp.wait()
            plsc.subcore_barrier()

            def fbody(j, _):
                for sl in range(NSL):
                    pn = (sl + NSL - 1) % NSL

                    @pl.when((j & (NSL - 1)) == sl)
                    def _():
                        @pl.when((j >= NSL) & (j - NSL < nblk))
                        def _():
                            pltpu.make_async_copy(
                                feat_hbm.at[pl.ds(0, GB)],
                                rows.at[pl.ds(GB * sl, GB)],
                                ssems[sl]).wait()

                        @pl.when(j < nblk)
                        def _():
                            pltpu.async_copy(
                                feat_hbm.at[stage_src.at[pl.ds(GB * j, GB)]],
                                rows.at[pl.ds(GB * sl, GB)], gsems[sl])

                        @pl.when((j >= 1) & (j <= nblk))
                        def _():
                            pltpu.make_async_copy(
                                feat_hbm.at[pl.ds(0, GB)],
                                rows.at[pl.ds(GB * pn, GB)],
                                gsems[pn]).wait()
                            for k in range(GB // _L):
                                idx16 = stage_seg[
                                    pl.ds(GB * (j - 1) + _L * k, _L)]
                                pltpu.async_copy(
                                    rows.at[pl.ds(GB * pn + _L * k, _L)],
                                    acc.at[idx16], ssems[pn], add=True)
                return 0

            lax.fori_loop(0, nblk + NSL, fbody, 0)
            plsc.subcore_barrier()

            pltpu.sync_copy(acc.at[pl.ds(s * RPS, RPS)],
                            out_hbm.at[pl.ds(lo + s * RPS, RPS)])

    return sc_fn(feat_ext, src, seg)


def _tc_attn(feat, G, emb, Wq, Wk, Wv, Wa, ba2, Wp, bp2):
    N, D = feat.shape
    R = emb.shape[0]
    W = G.shape[1]
    BN = 2000
    grid = N // BN
    nb = N // BN

    def body(feat_ref, g0, g1, g2, emb_ref, wq, wk, wv, wa, ba_ref, wp,
             bp_ref, out_ref):
        f = feat_ref[...]
        dn = (((1,), (0,)), ((), ()))
        dt = (((1,), (1,)), ((), ()))
        waq = lax.dot_general(wa[...], wq[...], dn,
                              preferred_element_type=jnp.float32)
        wak = lax.dot_general(wa[...], wk[...], dn,
                              preferred_element_type=jnp.float32)
        qa = lax.dot_general(f, waq, dt, preferred_element_type=jnp.float32)
        ss = []
        vs = []
        for r, g_ref in enumerate((g0, g1, g2)):
            g = g_ref[...]
            sums = g[:, :D]
            cnt = g[:, D:D + 1]
            km = (sums / jnp.maximum(cnt, 1.0)) * emb_ref[r][None, :]
            s_r = qa - lax.dot_general(km, wak, dt,
                                       preferred_element_type=jnp.float32)
            ss.append(s_r + ba_ref[...])
            vs.append(lax.dot_general(km, wv[...], dt,
                                      preferred_element_type=jnp.float32))
        m = jnp.maximum(jnp.maximum(ss[0], ss[1]), ss[2])
        es = [jnp.exp(s_r - m) for s_r in ss]
        den = es[0] + es[1] + es[2]
        o = (es[0] * vs[0] + es[1] * vs[1] + es[2] * vs[2]) / den
        out_ref[...] = (
            lax.dot_general(o, wp[...], dt, preferred_element_type=jnp.float32)
            + bp_ref[...] + f)

    def gview(r):
        return pl.BlockSpec((BN, W), lambda i, r=r: (nb * r + i, 0))

    wspec = pl.BlockSpec((D, D), lambda i: (0, 0))
    bspec = pl.BlockSpec((1, D), lambda i: (0, 0))
    return pl.pallas_call(
        body,
        grid=(grid,),
        in_specs=[
            pl.BlockSpec((BN, D), lambda i: (i, 0)),
            gview(0), gview(1), gview(2),
            pl.BlockSpec((R, D), lambda i: (0, 0)),
            wspec, wspec, wspec, wspec, bspec, wspec, bspec,
        ],
        out_specs=pl.BlockSpec((BN, D), lambda i: (i, 0)),
        out_shape=jax.ShapeDtypeStruct((N, D), jnp.float32),
    )(feat, G, G, G, emb, Wq, Wk, Wv, Wa, ba2, Wp, bp2)


def kernel(feat, edge_index, edge_type, emb, Wq, Wk, Wv, Wa, ba, Wp, bp):
    N, D = feat.shape
    R = emb.shape[0]
    W = D + _L
    src = edge_index[0]
    dst = edge_index[1]
    seg = edge_type * N + dst
    feat_ext = jnp.concatenate(
        [feat, jnp.ones((N, 1), jnp.float32), jnp.zeros((N, _L - 1),
                                                        jnp.float32)],
        axis=1)
    G = _sc_segsum(feat_ext, src, seg, R * N)
    return _tc_attn(feat, G, emb, Wq, Wk, Wv, Wa,
                    ba.reshape(1, -1), Wp, bp.reshape(1, -1))


# speedup vs baseline: 5.7851x; 1.0942x over previous
import functools

import jax
import jax.numpy as jnp
from jax import lax
from jax.experimental import pallas as pl
from jax.experimental.pallas import tpu as pltpu
from jax.experimental.pallas import tpu_sc as plsc

_NC = 2
_NS = 16
_L = 16


def _sc_segsum(feat_ext, src, seg, n_seg):
    n_rows, W = feat_ext.shape
    E = src.shape[0]
    CH = 4096
    NCHUNK = -(-n_seg // CH)
    NCHUNK = -(-NCHUNK // _NC) * _NC
    PASSES = NCHUNK // _NC
    GOUT = NCHUNK * CH
    EPC = E // _NS
    BE = 400
    NB = EPC // BE
    NV = BE // _L
    GB = 32
    GSH = GB.bit_length() - 1
    NSL = 4
    SCAP = EPC + 2 * GB
    RPS = CH // _NS
    DUMMY = CH

    mesh = plsc.VectorSubcoreMesh(core_axis_name="c", subcore_axis_name="s")

    @functools.partial(
        pl.kernel,
        out_type=jax.ShapeDtypeStruct((GOUT, W), jnp.float32),
        mesh=mesh,
        compiler_params=pltpu.CompilerParams(
            needs_layout_passes=False, use_tc_tiling_on_sc=False),
        scratch_types=[
            pltpu.VMEM((2, BE), jnp.int32),
            pltpu.VMEM((2, BE), jnp.int32),
            pltpu.VMEM((SCAP,), jnp.int32),
            pltpu.VMEM((SCAP,), jnp.int32),
            pltpu.VMEM((NSL * GB, W), jnp.float32),
            pltpu.VMEM((8, W), jnp.float32),
            pltpu.VMEM_SHARED((CH + _L, W), jnp.float32),
        ] + [pltpu.SemaphoreType.DMA] * (2 * NSL + 3),
    )
    def sc_fn(feat_hbm, src_hbm, seg_hbm, out_hbm,
              meta_src, meta_seg, stage_src, stage_seg, rows, zblk, acc,
              *sems):
        c = lax.axis_index("c")
        s = lax.axis_index("s")
        gsems = sems[:NSL]
        ssems = sems[NSL:2 * NSL]
        msems = sems[2 * NSL:2 * NSL + 2]
        zsem = sems[2 * NSL + 2]

        zv = jnp.zeros((_L,), jnp.float32)
        for i in range(8):
            for j in range(W // _L):
                zblk[i, _L * j:_L * (j + 1)] = zv

        def issue_meta(b, buf):
            base = s * EPC + b * BE
            pltpu.async_copy(src_hbm.at[pl.ds(base, BE)],
                             meta_src.at[buf], msems[buf])
            pltpu.async_copy(seg_hbm.at[pl.ds(base, BE)],
                             meta_seg.at[buf], msems[buf])

        def drain_meta(buf):
            pltpu.make_async_copy(src_hbm.at[pl.ds(0, BE)],
                                  meta_src.at[buf], msems[buf]).wait()
            pltpu.make_async_copy(src_hbm.at[pl.ds(0, BE)],
                                  meta_seg.at[buf], msems[buf]).wait()

        def issue_zero():
            return [
                pltpu.async_copy(zblk, acc.at[pl.ds(s * RPS + 8 * t, 8)],
                                 zsem)
                for t in range(RPS // 8)
            ]

        def _flush_at(j, gather, jmax):
            for sl in range(NSL):
                pn = (sl + NSL - 1) % NSL

                @pl.when((j & (NSL - 1)) == sl)
                def _():
                    @pl.when(j >= NSL)
                    def _():
                        pltpu.make_async_copy(
                            feat_hbm.at[pl.ds(0, GB)],
                            rows.at[pl.ds(GB * sl, GB)],
                            ssems[sl]).wait()

                    if gather:
                        pltpu.async_copy(
                            feat_hbm.at[stage_src.at[pl.ds(GB * j, GB)]],
                            rows.at[pl.ds(GB * sl, GB)], gsems[sl])

                    cond = (j >= 1) if jmax is None else ((j >= 1) &
                                                          (j <= jmax))

                    @pl.when(cond)
                    def _():
                        pltpu.make_async_copy(
                            feat_hbm.at[pl.ds(0, GB)],
                            rows.at[pl.ds(GB * pn, GB)],
                            gsems[pn]).wait()
                        for k in range(GB // _L):
                            idx16 = stage_seg[
                                pl.ds(GB * (j - 1) + _L * k, _L)]
                            pltpu.async_copy(
                                rows.at[pl.ds(GB * pn + _L * k, _L)],
                                acc.at[idx16], ssems[pn], add=True)

        def fbody_main(j, _):
            _flush_at(j, gather=True, jmax=None)
            return 0

        zcps = issue_zero()
        for p in range(PASSES):
            chunk = c * PASSES + p
            lo = chunk * CH
            issue_meta(0, 0)
            issue_meta(1, 1)

            def make_step(buf):
                def stepf(i, off):
                    s16 = meta_src[buf, pl.ds(_L * i, _L)]
                    g16 = meta_seg[buf, pl.ds(_L * i, _L)]
                    gl = g16 - lo
                    msk = (gl >= 0) & (gl < CH)
                    mi = msk.astype(jnp.int32)
                    incl = plsc.cumsum(mi)
                    dst = off + incl - mi
                    plsc.store_scatter(stage_src, [dst], s16, mask=msk)
                    plsc.store_scatter(stage_seg, [dst], gl, mask=msk)
                    return off + incl[_L - 1]
                return stepf

            drain_meta(0)
            off = lax.fori_loop(0, NV, make_step(0), jnp.int32(0))
            for cp in zcps:
                cp.wait()
            plsc.subcore_barrier()

            def scan_parity(bufi):
                def fn(carry):
                    off, b = carry

                    @pl.when(b + 1 < NB)
                    def _():
                        issue_meta(b + 1, 1 - bufi)

                    drain_meta(bufi)
                    return lax.fori_loop(0, NV, make_step(bufi), off)
                return fn

            def bbody(b, carry):
                off, done = carry
                new_done = off >> GSH
                lax.fori_loop(done, new_done, fbody_main, 0)
                off = lax.cond((b & 1) == 0, scan_parity(0), scan_parity(1),
                               (off, b))
                return (off, new_done)

            off, done = lax.fori_loop(1, NB, bbody, (off, jnp.int32(0)))

            base0 = off - (off & (GB - 1))
            for kk in range(GB // _L):
                base = base0 + _L * kk
                lane = base + lax.iota(jnp.int32, _L)
                m = lane < off
                sv = stage_src[pl.ds(base, _L)]
                gv = stage_seg[pl.ds(base, _L)]
                stage_src[pl.ds(base, _L)] = jnp.where(m, sv, 0)
                stage_seg[pl.ds(base, _L)] = jnp.where(m, gv, DUMMY)
            nblk = (off + GB - 1) >> GSH
            lax.fori_loop(done, nblk, fbody_main, 0)

            def fbody_tail(j, _):
                _flush_at(j, gather=False, jmax=nblk)
                return 0

            lax.fori_loop(nblk, nblk + NSL, fbody_tail, 0)
            plsc.subcore_barrier()

            pltpu.sync_copy(acc.at[pl.ds(s * RPS, RPS)],
                            out_hbm.at[pl.ds(lo + s * RPS, RPS)])
            if p + 1 < PASSES:
                zcps = issue_zero()

    return sc_fn(feat_ext, src, seg)


def _tc_attn(feat, G, emb, Wq, Wk, Wv, Wa, ba2, Wp, bp2):
    N, D = feat.shape
    R = emb.shape[0]
    W = G.shape[1]
    BN = 2000
    grid = N // BN
    nb = N // BN

    def body(feat_ref, g0, g1, g2, emb_ref, wq, wk, wv, wa, ba_ref, wp,
             bp_ref, out_ref):
        f = feat_ref[...]
        dn = (((1,), (0,)), ((), ()))
        dt = (((1,), (1,)), ((), ()))
        waq = lax.dot_general(wa[...], wq[...], dn,
                              preferred_element_type=jnp.float32)
        wak = lax.dot_general(wa[...], wk[...], dn,
                              preferred_element_type=jnp.float32)
        qa = lax.dot_general(f, waq, dt, preferred_element_type=jnp.float32)
        ss = []
        vs = []
        for r, g_ref in enumerate((g0, g1, g2)):
            g = g_ref[...]
            sums = g[:, :D]
            cnt = g[:, D:D + 1]
            km = (sums / jnp.maximum(cnt, 1.0)) * emb_ref[r][None, :]
            s_r = qa - lax.dot_general(km, wak, dt,
                                       preferred_element_type=jnp.float32)
            ss.append(s_r + ba_ref[...])
            vs.append(lax.dot_general(km, wv[...], dt,
                                      preferred_element_type=jnp.float32))
        m = jnp.maximum(jnp.maximum(ss[0], ss[1]), ss[2])
        es = [jnp.exp(s_r - m) for s_r in ss]
        den = es[0] + es[1] + es[2]
        o = (es[0] * vs[0] + es[1] * vs[1] + es[2] * vs[2]) / den
        out_ref[...] = (
            lax.dot_general(o, wp[...], dt, preferred_element_type=jnp.float32)
            + bp_ref[...] + f)

    def gview(r):
        return pl.BlockSpec((BN, W), lambda i, r=r: (nb * r + i, 0))

    wspec = pl.BlockSpec((D, D), lambda i: (0, 0))
    bspec = pl.BlockSpec((1, D), lambda i: (0, 0))
    return pl.pallas_call(
        body,
        grid=(grid,),
        in_specs=[
            pl.BlockSpec((BN, D), lambda i: (i, 0)),
            gview(0), gview(1), gview(2),
            pl.BlockSpec((R, D), lambda i: (0, 0)),
            wspec, wspec, wspec, wspec, bspec, wspec, bspec,
        ],
        out_specs=pl.BlockSpec((BN, D), lambda i: (i, 0)),
        out_shape=jax.ShapeDtypeStruct((N, D), jnp.float32),
    )(feat, G, G, G, emb, Wq, Wk, Wv, Wa, ba2, Wp, bp2)


def kernel(feat, edge_index, edge_type, emb, Wq, Wk, Wv, Wa, ba, Wp, bp):
    N, D = feat.shape
    R = emb.shape[0]
    W = D + _L
    src = edge_index[0]
    dst = edge_index[1]
    seg = edge_type * N + dst
    feat_ext = jnp.concatenate(
        [feat, jnp.ones((N, 1), jnp.float32), jnp.zeros((N, _L - 1),
                                                        jnp.float32)],
        axis=1)
    G = _sc_segsum(feat_ext, src, seg, R * N)
    return _tc_attn(feat, G, emb, Wq, Wk, Wv, Wa,
                    ba.reshape(1, -1), Wp, bp.reshape(1, -1))


# speedup vs baseline: 6.2818x; 1.0858x over previous
import functools

import jax
import jax.numpy as jnp
from jax import lax
from jax.experimental import pallas as pl
from jax.experimental.pallas import tpu as pltpu
from jax.experimental.pallas import tpu_sc as plsc

_NC = 2
_NS = 16
_L = 16


def _sc_segsum(feat_ext, src, seg, n_seg):
    n_rows, W = feat_ext.shape
    E = src.shape[0]
    CH = 3840
    NCHUNK = -(-n_seg // CH)
    NCHUNK = -(-NCHUNK // _NC) * _NC
    PASSES = NCHUNK // _NC
    GOUT = NCHUNK * CH
    EPC = E // _NS
    BE = 400
    NB = EPC // BE
    NV = BE // _L
    GB = 32
    GSH = GB.bit_length() - 1
    NSL = 4
    SCAP = EPC + 2 * GB
    RPS = CH // _NS
    DUMMY = CH

    mesh = plsc.VectorSubcoreMesh(core_axis_name="c", subcore_axis_name="s")

    @functools.partial(
        pl.kernel,
        out_type=(
            jax.ShapeDtypeStruct((GOUT // 8, 2, 8, 128), jnp.float32),
            jax.ShapeDtypeStruct((GOUT // 8, 8, _L), jnp.float32),
        ),
        mesh=mesh,
        compiler_params=pltpu.CompilerParams(
            needs_layout_passes=False, use_tc_tiling_on_sc=False),
        scratch_types=[
            pltpu.VMEM((2, BE), jnp.int32),
            pltpu.VMEM((2, BE), jnp.int32),
            pltpu.VMEM((SCAP,), jnp.int32),
            pltpu.VMEM((SCAP,), jnp.int32),
            pltpu.VMEM((NSL * GB, W), jnp.float32),
            pltpu.VMEM((8, W), jnp.float32),
            pltpu.VMEM_SHARED((CH + _L, W), jnp.float32),
        ] + [pltpu.SemaphoreType.DMA] * (2 * NSL + 3),
    )
    def sc_fn(feat_hbm, src_hbm, seg_hbm, g2_hbm, cnt_hbm,
              meta_src, meta_seg, stage_src, stage_seg, rows, zblk, acc,
              *sems):
        c = lax.axis_index("c")
        s = lax.axis_index("s")
        gsems = sems[:NSL]
        ssems = sems[NSL:2 * NSL]
        msems = sems[2 * NSL:2 * NSL + 2]
        zsem = sems[2 * NSL + 2]

        zv = jnp.zeros((_L,), jnp.float32)
        for i in range(8):
            for j in range(W // _L):
                zblk[i, _L * j:_L * (j + 1)] = zv

        def issue_meta(b, buf):
            base = s * EPC + b * BE
            pltpu.async_copy(src_hbm.at[pl.ds(base, BE)],
                             meta_src.at[buf], msems[buf])
            pltpu.async_copy(seg_hbm.at[pl.ds(base, BE)],
                             meta_seg.at[buf], msems[buf])

        def drain_meta(buf):
            pltpu.make_async_copy(src_hbm.at[pl.ds(0, BE)],
                                  meta_src.at[buf], msems[buf]).wait()
            pltpu.make_async_copy(src_hbm.at[pl.ds(0, BE)],
                                  meta_seg.at[buf], msems[buf]).wait()

        def issue_zero():
            return [
                pltpu.async_copy(zblk, acc.at[pl.ds(s * RPS + 8 * t, 8)],
                                 zsem)
                for t in range(RPS // 8)
            ]

        def _flush_at(j, gather, jmax):
            for sl in range(NSL):
                pn = (sl + NSL - 1) % NSL

                @pl.when((j & (NSL - 1)) == sl)
                def _():
                    @pl.when(j >= NSL)
                    def _():
                        pltpu.make_async_copy(
                            feat_hbm.at[pl.ds(0, GB)],
                            rows.at[pl.ds(GB * sl, GB)],
                            ssems[sl]).wait()

                    if gather:
                        pltpu.async_copy(
                            feat_hbm.at[stage_src.at[pl.ds(GB * j, GB)]],
                            rows.at[pl.ds(GB * sl, GB)], gsems[sl])

                    cond = (j >= 1) if jmax is None else ((j >= 1) &
                                                          (j <= jmax))

                    @pl.when(cond)
                    def _():
                        pltpu.make_async_copy(
                            feat_hbm.at[pl.ds(0, GB)],
                            rows.at[pl.ds(GB * pn, GB)],
                            gsems[pn]).wait()
                        for k in range(GB // _L):
                            idx16 = stage_seg[
                                pl.ds(GB * (j - 1) + _L * k, _L)]
                            pltpu.async_copy(
                                rows.at[pl.ds(GB * pn + _L * k, _L)],
                                acc.at[idx16], ssems[pn], add=True)

        def fbody_main(j, _):
            _flush_at(j, gather=True, jmax=None)
            return 0

        zcps = issue_zero()
        for p in range(PASSES):
            chunk = c * PASSES + p
            lo = chunk * CH
            issue_meta(0, 0)
            issue_meta(1, 1)

            def make_step(buf):
                def stepf(i, off):
                    s16 = meta_src[buf, pl.ds(_L * i, _L)]
                    g16 = meta_seg[buf, pl.ds(_L * i, _L)]
                    gl = g16 - lo
                    msk = (gl >= 0) & (gl < CH)
                    mi = msk.astype(jnp.int32)
                    incl = plsc.cumsum(mi)
                    dst = off + incl - mi
                    plsc.store_scatter(stage_src, [dst], s16, mask=msk)
                    plsc.store_scatter(stage_seg, [dst], gl, mask=msk)
                    return off + incl[_L - 1]
                return stepf

            drain_meta(0)
            off = lax.fori_loop(0, NV, make_step(0), jnp.int32(0))
            for cp in zcps:
                cp.wait()
            plsc.subcore_barrier()

            def scan_parity(bufi):
                def fn(carry):
                    off, b = carry

                    @pl.when(b + 1 < NB)
                    def _():
                        issue_meta(b + 1, 1 - bufi)

                    drain_meta(bufi)
                    return lax.fori_loop(0, NV, make_step(bufi), off)
                return fn

            def bbody(b, carry):
                off, done = carry
                new_done = off >> GSH
                lax.fori_loop(done, new_done, fbody_main, 0)
                off = lax.cond((b & 1) == 0, scan_parity(0), scan_parity(1),
                               (off, b))
                return (off, new_done)

            off, done = lax.fori_loop(1, NB, bbody, (off, jnp.int32(0)))

            base0 = off - (off & (GB - 1))
            for kk in range(GB // _L):
                base = base0 + _L * kk
                lane = base + lax.iota(jnp.int32, _L)
                m = lane < off
                sv = stage_src[pl.ds(base, _L)]
                gv = stage_seg[pl.ds(base, _L)]
                stage_src[pl.ds(base, _L)] = jnp.where(m, sv, 0)
                stage_seg[pl.ds(base, _L)] = jnp.where(m, gv, DUMMY)
            nblk = (off + GB - 1) >> GSH
            lax.fori_loop(done, nblk, fbody_main, 0)

            def fbody_tail(j, _):
                _flush_at(j, gather=False, jmax=nblk)
                return 0

            lax.fori_loop(nblk, nblk + NSL, fbody_tail, 0)
            plsc.subcore_barrier()

            r0 = s * RPS
            gr0 = (lo + s * RPS) // 8
            ccps = []
            for g in range(RPS // 8):
                for k in range(2):
                    ccps.append(pltpu.async_copy(
                        acc.at[pl.ds(r0 + 8 * g, 8), pl.ds(128 * k, 128)],
                        g2_hbm.at[gr0 + g, k], zsem))
                ccps.append(pltpu.async_copy(
                    acc.at[pl.ds(r0 + 8 * g, 8), pl.ds(2 * 128, _L)],
                    cnt_hbm.at[gr0 + g], zsem))
            for cp in ccps:
                cp.wait()
            if p + 1 < PASSES:
                zcps = issue_zero()

    g2, cnt = sc_fn(feat_ext, src, seg)
    return g2.reshape(GOUT, W - _L), cnt.reshape(GOUT, _L)


def _tc_attn(feat, G, C, emb, Wq, Wk, Wv, Wa, ba2, Wp, bp2):
    N, D = feat.shape
    R = emb.shape[0]
    BN = 2000
    grid = N // BN
    nb = N // BN

    def body(feat_ref, g0, g1, g2, c0, c1, c2, emb_ref, wq, wk, wv, wa,
             ba_ref, wp, bp_ref, out_ref):
        f = feat_ref[...]
        dn = (((1,), (0,)), ((), ()))
        dt = (((1,), (1,)), ((), ()))
        waq = lax.dot_general(wa[...], wq[...], dn,
                              preferred_element_type=jnp.float32)
        wak = lax.dot_general(wa[...], wk[...], dn,
                              preferred_element_type=jnp.float32)
        qa = lax.dot_general(f, waq, dt, preferred_element_type=jnp.float32)
        ss = []
        vs = []
        for r, (g_ref, c_ref) in enumerate(((g0, c0), (g1, c1), (g2, c2))):
            sums = g_ref[...]
            cnt = c_ref[...][:, 0:1]
            km = (sums / jnp.maximum(cnt, 1.0)) * emb_ref[r][None, :]
            s_r = qa - lax.dot_general(km, wak, dt,
                                       preferred_element_type=jnp.float32)
            ss.append(s_r + ba_ref[...])
            vs.append(lax.dot_general(km, wv[...], dt,
                                      preferred_element_type=jnp.float32))
        m = jnp.maximum(jnp.maximum(ss[0], ss[1]), ss[2])
        es = [jnp.exp(s_r - m) for s_r in ss]
        den = es[0] + es[1] + es[2]
        o = (es[0] * vs[0] + es[1] * vs[1] + es[2] * vs[2]) / den
        out_ref[...] = (
            lax.dot_general(o, wp[...], dt, preferred_element_type=jnp.float32)
            + bp_ref[...] + f)

    def gview(r):
        return pl.BlockSpec((BN, D), lambda i, r=r: (nb * r + i, 0))

    def cview(r):
        return pl.BlockSpec((BN, _L), lambda i, r=r: (nb * r + i, 0))

    wspec = pl.BlockSpec((D, D), lambda i: (0, 0))
    bspec = pl.BlockSpec((1, D), lambda i: (0, 0))
    return pl.pallas_call(
        body,
        grid=(grid,),
        in_specs=[
            pl.BlockSpec((BN, D), lambda i: (i, 0)),
            gview(0), gview(1), gview(2),
            cview(0), cview(1), cview(2),
            pl.BlockSpec((R, D), lambda i: (0, 0)),
            wspec, wspec, wspec, wspec, bspec, wspec, bspec,
        ],
        out_specs=pl.BlockSpec((BN, D), lambda i: (i, 0)),
        out_shape=jax.ShapeDtypeStruct((N, D), jnp.float32),
    )(feat, G, G, G, C, C, C, emb, Wq, Wk, Wv, Wa, ba2, Wp, bp2)


def kernel(feat, edge_index, edge_type, emb, Wq, Wk, Wv, Wa, ba, Wp, bp):
    N, D = feat.shape
    R = emb.shape[0]
    W = D + _L
    src = edge_index[0]
    dst = edge_index[1]
    seg = edge_type * N + dst
    feat_ext = jnp.concatenate(
        [feat, jnp.ones((N, 1), jnp.float32), jnp.zeros((N, _L - 1),
                                                        jnp.float32)],
        axis=1)
    G, C = _sc_segsum(feat_ext, src, seg, R * N)
    return _tc_attn(feat, G, C, emb, Wq, Wk, Wv, Wa,
                    ba.reshape(1, -1), Wp, bp.reshape(1, -1))


# speedup vs baseline: 6.5859x; 1.0484x over previous
import functools

import jax
import jax.numpy as jnp
from jax import lax
from jax.experimental import pallas as pl
from jax.experimental.pallas import tpu as pltpu
from jax.experimental.pallas import tpu_sc as plsc

_NC = 2
_NS = 16
_L = 16


def _sc_segsum(feat_ext, src, seg, n_seg):
    n_rows, W = feat_ext.shape
    E = src.shape[0]
    CH = 3840
    NCHUNK = -(-n_seg // CH)
    NCHUNK = -(-NCHUNK // _NC) * _NC
    PASSES = NCHUNK // _NC
    GOUT = NCHUNK * CH
    EPC = E // _NS
    BE = 400
    NB = EPC // BE
    NV = BE // _L
    GB = 32
    GSH = GB.bit_length() - 1
    NSL = 4
    SCAP = EPC + 2 * GB
    RPS = CH // _NS
    DUMMY = CH

    mesh = plsc.VectorSubcoreMesh(core_axis_name="c", subcore_axis_name="s")

    @functools.partial(
        pl.kernel,
        out_type=(
            jax.ShapeDtypeStruct((GOUT // 8, 2, 8, 128), jnp.float32),
            jax.ShapeDtypeStruct((GOUT // 8, 8, _L), jnp.float32),
        ),
        mesh=mesh,
        compiler_params=pltpu.CompilerParams(
            needs_layout_passes=False, use_tc_tiling_on_sc=False),
        scratch_types=[
            pltpu.VMEM((2, BE), jnp.int32),
            pltpu.VMEM((2, BE), jnp.int32),
            pltpu.VMEM((SCAP,), jnp.int32),
            pltpu.VMEM((SCAP,), jnp.int32),
            pltpu.VMEM((NSL * GB, W), jnp.float32),
            pltpu.VMEM((8, W), jnp.float32),
            pltpu.VMEM_SHARED((CH + _L, W), jnp.float32),
        ] + [pltpu.SemaphoreType.DMA] * (2 * NSL + 3),
    )
    def sc_fn(feat_hbm, src_hbm, seg_hbm, g2_hbm, cnt_hbm,
              meta_src, meta_seg, stage_src, stage_seg, rows, zblk, acc,
              *sems):
        c = lax.axis_index("c")
        s = lax.axis_index("s")
        gsems = sems[:NSL]
        ssems = sems[NSL:2 * NSL]
        msems = sems[2 * NSL:2 * NSL + 2]
        zsem = sems[2 * NSL + 2]

        zv = jnp.zeros((_L,), jnp.float32)
        for i in range(8):
            for j in range(W // _L):
                zblk[i, _L * j:_L * (j + 1)] = zv

        def issue_meta(b, buf):
            base = s * EPC + b * BE
            pltpu.async_copy(src_hbm.at[pl.ds(base, BE)],
                             meta_src.at[buf], msems[buf])
            pltpu.async_copy(seg_hbm.at[pl.ds(base, BE)],
                             meta_seg.at[buf], msems[buf])

        def drain_meta(buf):
            pltpu.make_async_copy(src_hbm.at[pl.ds(0, BE)],
                                  meta_src.at[buf], msems[buf]).wait()
            pltpu.make_async_copy(src_hbm.at[pl.ds(0, BE)],
                                  meta_seg.at[buf], msems[buf]).wait()

        def issue_zero():
            return [
                pltpu.async_copy(zblk, acc.at[pl.ds(s * RPS + 8 * t, 8)],
                                 zsem)
                for t in range(RPS // 8)
            ]

        def _flush_at(j, gather, jmax):
            for sl in range(NSL):
                pn = (sl + NSL - 1) % NSL

                @pl.when((j & (NSL - 1)) == sl)
                def _():
                    @pl.when(j >= NSL)
                    def _():
                        pltpu.make_async_copy(
                            feat_hbm.at[pl.ds(0, GB)],
                            rows.at[pl.ds(GB * sl, GB)],
                            ssems[sl]).wait()

                    if gather:
                        pltpu.async_copy(
                            feat_hbm.at[stage_src.at[pl.ds(GB * j, GB)]],
                            rows.at[pl.ds(GB * sl, GB)], gsems[sl])

                    cond = (j >= 1) if jmax is None else ((j >= 1) &
                                                          (j <= jmax))

                    @pl.when(cond)
                    def _():
                        pltpu.make_async_copy(
                            feat_hbm.at[pl.ds(0, GB)],
                            rows.at[pl.ds(GB * pn, GB)],
                            gsems[pn]).wait()
                        for k in range(GB // _L):
                            idx16 = stage_seg[
                                pl.ds(GB * (j - 1) + _L * k, _L)]
                            pltpu.async_copy(
                                rows.at[pl.ds(GB * pn + _L * k, _L)],
                                acc.at[idx16], ssems[pn], add=True)

        def fbody_main(j, _):
            _flush_at(j, gather=True, jmax=None)
            return 0

        zcps = issue_zero()
        for p in range(PASSES):
            chunk = c * PASSES + p
            lo = chunk * CH
            issue_meta(0, 0)
            issue_meta(1, 1)

            def make_step(buf):
                def stepf(i, off):
                    s16 = meta_src[buf, pl.ds(_L * i, _L)]
                    g16 = meta_seg[buf, pl.ds(_L * i, _L)]
                    gl = g16 - lo
                    msk = (gl >= 0) & (gl < CH)
                    mi = msk.astype(jnp.int32)
                    incl = plsc.cumsum(mi)
                    dst = off + incl - mi
                    plsc.store_scatter(stage_src, [dst], s16, mask=msk)
                    plsc.store_scatter(stage_seg, [dst], gl, mask=msk)
                    return off + incl[_L - 1]
                return stepf

            drain_meta(0)
            off = lax.fori_loop(0, NV, make_step(0), jnp.int32(0))
            for cp in zcps:
                cp.wait()
            plsc.subcore_barrier()

            def scan_parity(bufi):
                def fn(carry):
                    off, b = carry

                    @pl.when(b + 1 < NB)
                    def _():
                        issue_meta(b + 1, 1 - bufi)

                    drain_meta(bufi)
                    return lax.fori_loop(0, NV, make_step(bufi), off)
                return fn

            def bbody(b, carry):
                off, done = carry
                new_done = off >> GSH
                lax.fori_loop(done, new_done, fbody_main, 0)
                off = lax.cond((b & 1) == 0, scan_parity(0), scan_parity(1),
                               (off, b))
                return (off, new_done)

            off, done = lax.fori_loop(1, NB, bbody, (off, jnp.int32(0)))

            base0 = off - (off & (GB - 1))
            for kk in range(GB // _L):
                base = base0 + _L * kk
                lane = base + lax.iota(jnp.int32, _L)
                m = lane < off
                sv = stage_src[pl.ds(base, _L)]
                gv = stage_seg[pl.ds(base, _L)]
                stage_src[pl.ds(base, _L)] = jnp.where(m, sv, 0)
                stage_seg[pl.ds(base, _L)] = jnp.where(m, gv, DUMMY)
            nblk = (off + GB - 1) >> GSH
            lax.fori_loop(done, nblk, fbody_main, 0)

            def fbody_tail(j, _):
                _flush_at(j, gather=False, jmax=nblk)
                return 0

            lax.fori_loop(nblk, nblk + NSL, fbody_tail, 0)
            plsc.subcore_barrier()

            r0 = s * RPS
            gr0 = (lo + s * RPS) // 8
            ccps = []
            for g in range(RPS // 8):
                for k in range(2):
                    ccps.append(pltpu.async_copy(
                        acc.at[pl.ds(r0 + 8 * g, 8), pl.ds(128 * k, 128)],
                        g2_hbm.at[gr0 + g, k], zsem))
                ccps.append(pltpu.async_copy(
                    acc.at[pl.ds(r0 + 8 * g, 8), pl.ds(2 * 128, _L)],
                    cnt_hbm.at[gr0 + g], zsem))
            for cp in ccps:
                cp.wait()
            if p + 1 < PASSES:
                zcps = issue_zero()

    g2, cnt = sc_fn(feat_ext, src, seg)
    return (g2.transpose(0, 2, 1, 3).reshape(GOUT, W - _L),
            cnt.reshape(GOUT, _L))


def _tc_attn(feat, G, C, emb, Wq, Wk, Wv, Wa, ba2, Wp, bp2):
    N, D = feat.shape
    R = emb.shape[0]
    BN = 2000
    grid = N // BN
    nb = N // BN

    def body(feat_ref, g0, g1, g2, c0, c1, c2, emb_ref, wq, wk, wv, wa,
             ba_ref, wp, bp_ref, out_ref):
        f = feat_ref[...]
        dn = (((1,), (0,)), ((), ()))
        dt = (((1,), (1,)), ((), ()))
        waq = lax.dot_general(wa[...], wq[...], dn,
                              preferred_element_type=jnp.float32)
        wak = lax.dot_general(wa[...], wk[...], dn,
                              preferred_element_type=jnp.float32)
        qa = lax.dot_general(f, waq, dt, preferred_element_type=jnp.float32)
        ss = []
        vs = []
        for r, (g_ref, c_ref) in enumerate(((g0, c0), (g1, c1), (g2, c2))):
            sums = g_ref[...]
            cnt = c_ref[...][:, 0:1]
            km = (sums / jnp.maximum(cnt, 1.0)) * emb_ref[r][None, :]
            s_r = qa - lax.dot_general(km, wak, dt,
                                       preferred_element_type=jnp.float32)
            ss.append(s_r + ba_ref[...])
            vs.append(lax.dot_general(km, wv[...], dt,
                                      preferred_element_type=jnp.float32))
        m = jnp.maximum(jnp.maximum(ss[0], ss[1]), ss[2])
        es = [jnp.exp(s_r - m) for s_r in ss]
        den = es[0] + es[1] + es[2]
        o = (es[0] * vs[0] + es[1] * vs[1] + es[2] * vs[2]) / den
        out_ref[...] = (
            lax.dot_general(o, wp[...], dt, preferred_element_type=jnp.float32)
            + bp_ref[...] + f)

    def gview(r):
        return pl.BlockSpec((BN, D), lambda i, r=r: (nb * r + i, 0))

    def cview(r):
        return pl.BlockSpec((BN, _L), lambda i, r=r: (nb * r + i, 0))

    wspec = pl.BlockSpec((D, D), lambda i: (0, 0))
    bspec = pl.BlockSpec((1, D), lambda i: (0, 0))
    return pl.pallas_call(
        body,
        grid=(grid,),
        in_specs=[
            pl.BlockSpec((BN, D), lambda i: (i, 0)),
            gview(0), gview(1), gview(2),
            cview(0), cview(1), cview(2),
            pl.BlockSpec((R, D), lambda i: (0, 0)),
            wspec, wspec, wspec, wspec, bspec, wspec, bspec,
        ],
        out_specs=pl.BlockSpec((BN, D), lambda i: (i, 0)),
        out_shape=jax.ShapeDtypeStruct((N, D), jnp.float32),
    )(feat, G, G, G, C, C, C, emb, Wq, Wk, Wv, Wa, ba2, Wp, bp2)


def kernel(feat, edge_index, edge_type, emb, Wq, Wk, Wv, Wa, ba, Wp, bp):
    N, D = feat.shape
    R = emb.shape[0]
    W = D + _L
    src = edge_index[0]
    dst = edge_index[1]
    seg = edge_type * N + dst
    feat_ext = jnp.concatenate(
        [feat, jnp.ones((N, 1), jnp.float32), jnp.zeros((N, _L - 1),
                                                        jnp.float32)],
        axis=1)
    G, C = _sc_segsum(feat_ext, src, seg, R * N)
    return _tc_attn(feat, G, C, emb, Wq, Wk, Wv, Wa,
                    ba.reshape(1, -1), Wp, bp.reshape(1, -1))


# speedup vs baseline: 6.9510x; 1.0554x over previous
import functools

import jax
import jax.numpy as jnp
from jax import lax
from jax.experimental import pallas as pl
from jax.experimental.pallas import tpu as pltpu
from jax.experimental.pallas import tpu_sc as plsc

_NC = 2
_NS = 16
_L = 16


def _sc_segsum(feat_ext, src, seg, n_seg):
    n_rows, W = feat_ext.shape
    E = src.shape[0]
    CH = 5120
    NCHUNK = -(-n_seg // CH)
    NCHUNK = -(-NCHUNK // _NC) * _NC
    PASSES = NCHUNK // _NC
    GOUT = NCHUNK * CH
    EPC = E // _NS
    BE = 400
    NB = EPC // BE
    NV = BE // _L
    GB = 32
    GSH = GB.bit_length() - 1
    NSL = 4
    SCAP = 1024
    SMSK = SCAP - 1
    RBLK = SCAP // GB
    RPS = CH // _NS
    DUMMY = CH

    mesh = plsc.VectorSubcoreMesh(core_axis_name="c", subcore_axis_name="s")

    @functools.partial(
        pl.kernel,
        out_type=(
            jax.ShapeDtypeStruct((GOUT // 8, 2, 8, 128), jnp.float32),
            jax.ShapeDtypeStruct((GOUT // 8, 8, _L), jnp.float32),
        ),
        mesh=mesh,
        compiler_params=pltpu.CompilerParams(
            needs_layout_passes=False, use_tc_tiling_on_sc=False),
        scratch_types=[
            pltpu.VMEM((2, BE), jnp.int32),
            pltpu.VMEM((2, BE), jnp.int32),
            pltpu.VMEM((SCAP,), jnp.int32),
            pltpu.VMEM((SCAP,), jnp.int32),
            pltpu.VMEM((NSL * GB, W), jnp.float32),
            pltpu.VMEM((8, W), jnp.float32),
            pltpu.VMEM_SHARED((CH + _L, W), jnp.float32),
        ] + [pltpu.SemaphoreType.DMA] * (2 * NSL + 3),
    )
    def sc_fn(feat_hbm, src_hbm, seg_hbm, g2_hbm, cnt_hbm,
              meta_src, meta_seg, stage_src, stage_seg, rows, zblk, acc,
              *sems):
        c = lax.axis_index("c")
        s = lax.axis_index("s")
        gsems = sems[:NSL]
        ssems = sems[NSL:2 * NSL]
        msems = sems[2 * NSL:2 * NSL + 2]
        zsem = sems[2 * NSL + 2]

        zv = jnp.zeros((_L,), jnp.float32)
        for i in range(8):
            for j in range(W // _L):
                zblk[i, _L * j:_L * (j + 1)] = zv

        def issue_meta(b, buf):
            base = s * EPC + b * BE
            pltpu.async_copy(src_hbm.at[pl.ds(base, BE)],
                             meta_src.at[buf], msems[buf])
            pltpu.async_copy(seg_hbm.at[pl.ds(base, BE)],
                             meta_seg.at[buf], msems[buf])

        def drain_meta(buf):
            pltpu.make_async_copy(src_hbm.at[pl.ds(0, BE)],
                                  meta_src.at[buf], msems[buf]).wait()
            pltpu.make_async_copy(src_hbm.at[pl.ds(0, BE)],
                                  meta_seg.at[buf], msems[buf]).wait()

        def issue_zero():
            return [
                pltpu.async_copy(zblk, acc.at[pl.ds(s * RPS + 8 * t, 8)],
                                 zsem)
                for t in range(RPS // 8)
            ]

        def _flush_at(j, gather, jmax):
            for sl in range(NSL):
                pn = (sl + NSL - 1) % NSL

                @pl.when((j & (NSL - 1)) == sl)
                def _():
                    @pl.when(j >= NSL)
                    def _():
                        pltpu.make_async_copy(
                            feat_hbm.at[pl.ds(0, GB)],
                            rows.at[pl.ds(GB * sl, GB)],
                            ssems[sl]).wait()

                    if gather:
                        jr = GB * (j & (RBLK - 1))
                        pltpu.async_copy(
                            feat_hbm.at[stage_src.at[pl.ds(jr, GB)]],
                            rows.at[pl.ds(GB * sl, GB)], gsems[sl])

                    cond = (j >= 1) if jmax is None else ((j >= 1) &
                                                          (j <= jmax))

                    @pl.when(cond)
                    def _():
                        pltpu.make_async_copy(
                            feat_hbm.at[pl.ds(0, GB)],
                            rows.at[pl.ds(GB * pn, GB)],
                            gsems[pn]).wait()
                        pr = GB * ((j - 1) & (RBLK - 1))
                        for k in range(GB // _L):
                            idx16 = stage_seg[pl.ds(pr + _L * k, _L)]
                            pltpu.async_copy(
                                rows.at[pl.ds(GB * pn + _L * k, _L)],
                                acc.at[idx16], ssems[pn], add=True)

        def fbody_main(j, _):
            _flush_at(j, gather=True, jmax=None)
            return 0

        zcps = issue_zero()
        for p in range(PASSES):
            chunk = c * PASSES + p
            lo = chunk * CH
            issue_meta(0, 0)
            issue_meta(1, 1)

            def make_step(buf):
                def stepf(i, off):
                    s16 = meta_src[buf, pl.ds(_L * i, _L)]
                    g16 = meta_seg[buf, pl.ds(_L * i, _L)]
                    gl = g16 - lo
                    msk = (gl >= 0) & (gl < CH)
                    mi = msk.astype(jnp.int32)
                    incl = plsc.cumsum(mi)
                    dst = (off + incl - mi) & SMSK
                    plsc.store_scatter(stage_src, [dst], s16, mask=msk)
                    plsc.store_scatter(stage_seg, [dst], gl, mask=msk)
                    return off + incl[_L - 1]
                return stepf

            drain_meta(0)
            off = lax.fori_loop(0, NV, make_step(0), jnp.int32(0))
            for cp in zcps:
                cp.wait()
            plsc.subcore_barrier()

            def scan_parity(bufi):
                def fn(carry):
                    off, b = carry

                    @pl.when(b + 1 < NB)
                    def _():
                        issue_meta(b + 1, 1 - bufi)

                    drain_meta(bufi)
                    return lax.fori_loop(0, NV, make_step(bufi), off)
                return fn

            def bbody(b, carry):
                off, done = carry
                new_done = off >> GSH
                lax.fori_loop(done, new_done, fbody_main, 0)
                off = lax.cond((b & 1) == 0, scan_parity(0), scan_parity(1),
                               (off, b))
                return (off, new_done)

            off, done = lax.fori_loop(1, NB, bbody, (off, jnp.int32(0)))

            rnd = (off + GB - 1) & ~jnp.int32(GB - 1)
            for kk in range(GB // _L):
                pos = off + _L * kk + lax.iota(jnp.int32, _L)
                m = pos < rnd
                plsc.store_scatter(stage_src, [pos & SMSK],
                                   jnp.zeros((_L,), jnp.int32), mask=m)
                plsc.store_scatter(stage_seg, [pos & SMSK],
                                   jnp.full((_L,), DUMMY, jnp.int32), mask=m)
            nblk = (off + GB - 1) >> GSH
            lax.fori_loop(done, nblk, fbody_main, 0)

            def fbody_tail(j, _):
                _flush_at(j, gather=False, jmax=nblk)
                return 0

            lax.fori_loop(nblk, nblk + NSL, fbody_tail, 0)
            plsc.subcore_barrier()

            r0 = s * RPS
            gr0 = (lo + s * RPS) // 8
            ccps = []
            for g in range(RPS // 8):
                for k in range(2):
                    ccps.append(pltpu.async_copy(
                        acc.at[pl.ds(r0 + 8 * g, 8), pl.ds(128 * k, 128)],
                        g2_hbm.at[gr0 + g, k], zsem))
                ccps.append(pltpu.async_copy(
                    acc.at[pl.ds(r0 + 8 * g, 8), pl.ds(2 * 128, _L)],
                    cnt_hbm.at[gr0 + g], zsem))
            for cp in ccps:
                cp.wait()
            if p + 1 < PASSES:
                zcps = issue_zero()

    g2, cnt = sc_fn(feat_ext, src, seg)
    return (g2.transpose(0, 2, 1, 3).reshape(GOUT, W - _L),
            cnt.reshape(GOUT, _L))


def _tc_attn(feat, G, C, emb, Wq, Wk, Wv, Wa, ba2, Wp, bp2):
    N, D = feat.shape
    R = emb.shape[0]
    BN = 2000
    grid = N // BN
    nb = N // BN

    def body(feat_ref, g0, g1, g2, c0, c1, c2, emb_ref, wq, wk, wv, wa,
             ba_ref, wp, bp_ref, out_ref):
        f = feat_ref[...]
        dn = (((1,), (0,)), ((), ()))
        dt = (((1,), (1,)), ((), ()))
        waq = lax.dot_general(wa[...], wq[...], dn,
                              preferred_element_type=jnp.float32)
        wak = lax.dot_general(wa[...], wk[...], dn,
                              preferred_element_type=jnp.float32)
        qa = lax.dot_general(f, waq, dt, preferred_element_type=jnp.float32)
        ss = []
        vs = []
        for r, (g_ref, c_ref) in enumerate(((g0, c0), (g1, c1), (g2, c2))):
            sums = g_ref[...]
            cnt = c_ref[...][:, 0:1]
            km = (sums / jnp.maximum(cnt, 1.0)) * emb_ref[r][None, :]
            s_r = qa - lax.dot_general(km, wak, dt,
                                       preferred_element_type=jnp.float32)
            ss.append(s_r + ba_ref[...])
            vs.append(lax.dot_general(km, wv[...], dt,
                                      preferred_element_type=jnp.float32))
        m = jnp.maximum(jnp.maximum(ss[0], ss[1]), ss[2])
        es = [jnp.exp(s_r - m) for s_r in ss]
        den = es[0] + es[1] + es[2]
        o = (es[0] * vs[0] + es[1] * vs[1] + es[2] * vs[2]) / den
        out_ref[...] = (
            lax.dot_general(o, wp[...], dt, preferred_element_type=jnp.float32)
            + bp_ref[...] + f)

    def gview(r):
        return pl.BlockSpec((BN, D), lambda i, r=r: (nb * r + i, 0))

    def cview(r):
        return pl.BlockSpec((BN, _L), lambda i, r=r: (nb * r + i, 0))

    wspec = pl.BlockSpec((D, D), lambda i: (0, 0))
    bspec = pl.BlockSpec((1, D), lambda i: (0, 0))
    return pl.pallas_call(
        body,
        grid=(grid,),
        in_specs=[
            pl.BlockSpec((BN, D), lambda i: (i, 0)),
            gview(0), gview(1), gview(2),
            cview(0), cview(1), cview(2),
            pl.BlockSpec((R, D), lambda i: (0, 0)),
            wspec, wspec, wspec, wspec, bspec, wspec, bspec,
        ],
        out_specs=pl.BlockSpec((BN, D), lambda i: (i, 0)),
        out_shape=jax.ShapeDtypeStruct((N, D), jnp.float32),
    )(feat, G, G, G, C, C, C, emb, Wq, Wk, Wv, Wa, ba2, Wp, bp2)


def kernel(feat, edge_index, edge_type, emb, Wq, Wk, Wv, Wa, ba, Wp, bp):
    N, D = feat.shape
    R = emb.shape[0]
    W = D + _L
    src = edge_index[0]
    dst = edge_index[1]
    seg = edge_type * N + dst
    feat_ext = jnp.concatenate(
        [feat, jnp.ones((N, 1), jnp.float32), jnp.zeros((N, _L - 1),
                                                        jnp.float32)],
        axis=1)
    G, C = _sc_segsum(feat_ext, src, seg, R * N)
    return _tc_attn(feat, G, C, emb, Wq, Wk, Wv, Wa,
                    ba.reshape(1, -1), Wp, bp.reshape(1, -1))


# speedup vs baseline: 7.5617x; 1.0879x over previous
import functools

import jax
import jax.numpy as jnp
from jax import lax
from jax.experimental import pallas as pl
from jax.experimental.pallas import tpu as pltpu
from jax.experimental.pallas import tpu_sc as plsc

_NC = 2
_NS = 16
_L = 16


def _sc_segsum(featrows, src, seg, n_seg):
    E = src.shape[0]
    CH = 5120
    NCHUNK = -(-n_seg // CH)
    NCHUNK = -(-NCHUNK // _NC) * _NC
    PASSES = NCHUNK // _NC
    GOUT = NCHUNK * CH
    EPC = E // _NS
    BE = 400
    NB = EPC // BE
    NV = BE // _L
    GB = 32
    GSH = GB.bit_length() - 1
    NSL = 4
    SCAP = 1024
    SMSK = SCAP - 1
    RBLK = SCAP // GB
    RPS = CH // _NS
    DUMMY = CH

    mesh = plsc.VectorSubcoreMesh(core_axis_name="c", subcore_axis_name="s")

    @functools.partial(
        pl.kernel,
        out_type=(
            jax.ShapeDtypeStruct((GOUT // 8, 2, 8, 128), jnp.float32),
            jax.ShapeDtypeStruct((GOUT // 8, 8, _L), jnp.float32),
        ),
        mesh=mesh,
        compiler_params=pltpu.CompilerParams(
            needs_layout_passes=False, use_tc_tiling_on_sc=False),
        scratch_types=[
            pltpu.VMEM((2, BE), jnp.int32),
            pltpu.VMEM((2, BE), jnp.int32),
            pltpu.VMEM((SCAP,), jnp.int32),
            pltpu.VMEM((SCAP,), jnp.int32),
            pltpu.VMEM((SCAP,), jnp.int32),
            pltpu.VMEM((NSL * GB, 128), jnp.float32),
            pltpu.VMEM((NSL * GB, 128), jnp.float32),
            pltpu.VMEM((GB, _L), jnp.float32),
            pltpu.VMEM((8, 128), jnp.float32),
            pltpu.VMEM((8, _L), jnp.float32),
            pltpu.VMEM_SHARED((CH + _L, 128), jnp.float32),
            pltpu.VMEM_SHARED((CH + _L, 128), jnp.float32),
            pltpu.VMEM_SHARED((CH + _L, _L), jnp.float32),
        ] + [pltpu.SemaphoreType.DMA] * (2 * NSL + 3),
    )
    def sc_fn(feat_hbm, src_hbm, seg_hbm, g2_hbm, cnt_hbm,
              meta_src, meta_seg, stage_a, stage_b, stage_seg,
              rows_a, rows_b, ones, zblk, zcnt, acc_a, acc_b, acc_cnt,
              *sems):
        c = lax.axis_index("c")
        s = lax.axis_index("s")
        gsems = sems[:NSL]
        ssems = sems[NSL:2 * NSL]
        msems = sems[2 * NSL:2 * NSL + 2]
        zsem = sems[2 * NSL + 2]

        zv = jnp.zeros((_L,), jnp.float32)
        ov = jnp.ones((_L,), jnp.float32)
        for i in range(8):
            for j in range(128 // _L):
                zblk[i, _L * j:_L * (j + 1)] = zv
            zcnt[i, 0:_L] = zv
        for i in range(GB):
            ones[i, 0:_L] = ov

        def issue_meta(b, buf):
            base = s * EPC + b * BE
            pltpu.async_copy(src_hbm.at[pl.ds(base, BE)],
                             meta_src.at[buf], msems[buf])
            pltpu.async_copy(seg_hbm.at[pl.ds(base, BE)],
                             meta_seg.at[buf], msems[buf])

        def drain_meta(buf):
            pltpu.make_async_copy(src_hbm.at[pl.ds(0, BE)],
                                  meta_src.at[buf], msems[buf]).wait()
            pltpu.make_async_copy(src_hbm.at[pl.ds(0, BE)],
                                  meta_seg.at[buf], msems[buf]).wait()

        def issue_zero():
            cps = []
            for t in range(RPS // 8):
                d = pl.ds(s * RPS + 8 * t, 8)
                cps.append(pltpu.async_copy(zblk, acc_a.at[d], zsem))
                cps.append(pltpu.async_copy(zblk, acc_b.at[d], zsem))
                cps.append(pltpu.async_copy(zcnt, acc_cnt.at[d], zsem))
            return cps

        def _flush_at(j, gather, jmax):
            for sl in range(NSL):
                pn = (sl + NSL - 1) % NSL

                @pl.when((j & (NSL - 1)) == sl)
                def _():
                    @pl.when(j >= NSL)
                    def _():
                        pltpu.make_async_copy(
                            feat_hbm.at[pl.ds(0, GB)],
                            rows_a.at[pl.ds(GB * sl, GB)],
                            ssems[sl]).wait()
                        pltpu.make_async_copy(
                            feat_hbm.at[pl.ds(0, GB)],
                            rows_b.at[pl.ds(GB * sl, GB)],
                            ssems[sl]).wait()
                        pltpu.make_async_copy(
                            feat_hbm.at[pl.ds(0, GB), pl.ds(0, _L)],
                            ones, ssems[sl]).wait()

                    if gather:
                        jr = GB * (j & (RBLK - 1))
                        pltpu.async_copy(
                            feat_hbm.at[stage_a.at[pl.ds(jr, GB)]],
                            rows_a.at[pl.ds(GB * sl, GB)], gsems[sl])
                        pltpu.async_copy(
                            feat_hbm.at[stage_b.at[pl.ds(jr, GB)]],
                            rows_b.at[pl.ds(GB * sl, GB)], gsems[sl])

                    cond = (j >= 1) if jmax is None else ((j >= 1) &
                                                          (j <= jmax))

                    @pl.when(cond)
                    def _():
                        pltpu.make_async_copy(
                            feat_hbm.at[pl.ds(0, GB)],
                            rows_a.at[pl.ds(GB * pn, GB)],
                            gsems[pn]).wait()
                        pltpu.make_async_copy(
                            feat_hbm.at[pl.ds(0, GB)],
                            rows_b.at[pl.ds(GB * pn, GB)],
                            gsems[pn]).wait()
                        pr = GB * ((j - 1) & (RBLK - 1))
                        for k in range(GB // _L):
                            idx16 = stage_seg[pl.ds(pr + _L * k, _L)]
                            pltpu.async_copy(
                                rows_a.at[pl.ds(GB * pn + _L * k, _L)],
                                acc_a.at[idx16], ssems[pn], add=True)
                            pltpu.async_copy(
                                rows_b.at[pl.ds(GB * pn + _L * k, _L)],
                                acc_b.at[idx16], ssems[pn], add=True)
                            pltpu.async_copy(
                                ones.at[pl.ds(_L * k, _L)],
                                acc_cnt.at[idx16], ssems[pn], add=True)

        def fbody_main(j, _):
            _flush_at(j, gather=True, jmax=None)
            return 0

        zcps = issue_zero()
        for p in range(PASSES):
            chunk = c * PASSES + p
            lo = chunk * CH
            issue_meta(0, 0)
            issue_meta(1, 1)

            def make_step(buf):
                def stepf(i, off):
                    s16 = meta_src[buf, pl.ds(_L * i, _L)]
                    g16 = meta_seg[buf, pl.ds(_L * i, _L)]
                    gl = g16 - lo
                    msk = (gl >= 0) & (gl < CH)
                    mi = msk.astype(jnp.int32)
                    incl = plsc.cumsum(mi)
                    dst = (off + incl - mi) & SMSK
                    ia = s16 + (s16 & jnp.int32(-8))
                    plsc.store_scatter(stage_a, [dst], ia, mask=msk)
                    plsc.store_scatter(stage_b, [dst], ia + 8, mask=msk)
                    plsc.store_scatter(stage_seg, [dst], gl, mask=msk)
                    return off + incl[_L - 1]
                return stepf

            drain_meta(0)
            off = lax.fori_loop(0, NV, make_step(0), jnp.int32(0))
            for cp in zcps:
                cp.wait()
            plsc.subcore_barrier()

            def scan_parity(bufi):
                def fn(carry):
                    off, b = carry

                    @pl.when(b + 1 < NB)
                    def _():
                        issue_meta(b + 1, 1 - bufi)

                    drain_meta(bufi)
                    return lax.fori_loop(0, NV, make_step(bufi), off)
                return fn

            def bbody(b, carry):
                off, done = carry
                new_done = off >> GSH
                lax.fori_loop(done, new_done, fbody_main, 0)
                off = lax.cond((b & 1) == 0, scan_parity(0), scan_parity(1),
                               (off, b))
                return (off, new_done)

            off, done = lax.fori_loop(1, NB, bbody, (off, jnp.int32(0)))

            rnd = (off + GB - 1) & ~jnp.int32(GB - 1)
            for kk in range(GB // _L):
                pos = off + _L * kk + lax.iota(jnp.int32, _L)
                m = pos < rnd
                plsc.store_scatter(stage_a, [pos & SMSK],
                                   jnp.zeros((_L,), jnp.int32), mask=m)
                plsc.store_scatter(stage_b, [pos & SMSK],
                                   jnp.full((_L,), 8, jnp.int32), mask=m)
                plsc.store_scatter(stage_seg, [pos & SMSK],
                                   jnp.full((_L,), DUMMY, jnp.int32), mask=m)
            nblk = (off + GB - 1) >> GSH
            lax.fori_loop(done, nblk, fbody_main, 0)

            def fbody_tail(j, _):
                _flush_at(j, gather=False, jmax=nblk)
                return 0

            lax.fori_loop(nblk, nblk + NSL, fbody_tail, 0)
            plsc.subcore_barrier()

            r0 = s * RPS
            gr0 = (lo + s * RPS) // 8
            ccps = []
            for g in range(RPS // 8):
                d = pl.ds(r0 + 8 * g, 8)
                ccps.append(pltpu.async_copy(
                    acc_a.at[d], g2_hbm.at[gr0 + g, 0], zsem))
                ccps.append(pltpu.async_copy(
                    acc_b.at[d], g2_hbm.at[gr0 + g, 1], zsem))
                ccps.append(pltpu.async_copy(
                    acc_cnt.at[d], cnt_hbm.at[gr0 + g], zsem))
            for cp in ccps:
                cp.wait()
            if p + 1 < PASSES:
                zcps = issue_zero()

    g2, cnt = sc_fn(featrows, src, seg)
    return (g2.transpose(0, 2, 1, 3).reshape(GOUT, 2 * 128),
            cnt.reshape(GOUT, _L))


def _tc_attn(feat, G, C, emb, Wq, Wk, Wv, Wa, ba2, Wp, bp2):
    N, D = feat.shape
    R = emb.shape[0]
    BN = 2000
    grid = N // BN
    nb = N // BN

    def body(feat_ref, g0, g1, g2, c0, c1, c2, emb_ref, wq, wk, wv, wa,
             ba_ref, wp, bp_ref, out_ref):
        f = feat_ref[...]
        dn = (((1,), (0,)), ((), ()))
        dt = (((1,), (1,)), ((), ()))
        waq = lax.dot_general(wa[...], wq[...], dn,
                              preferred_element_type=jnp.float32)
        wak = lax.dot_general(wa[...], wk[...], dn,
                              preferred_element_type=jnp.float32)
        qa = lax.dot_general(f, waq, dt, preferred_element_type=jnp.float32)
        ss = []
        vs = []
        for r, (g_ref, c_ref) in enumerate(((g0, c0), (g1, c1), (g2, c2))):
            sums = g_ref[...]
            cnt = c_ref[...][:, 0:1]
            km = (sums / jnp.maximum(cnt, 1.0)) * emb_ref[r][None, :]
            s_r = qa - lax.dot_general(km, wak, dt,
                                       preferred_element_type=jnp.float32)
            ss.append(s_r + ba_ref[...])
            vs.append(lax.dot_general(km, wv[...], dt,
                                      preferred_element_type=jnp.float32))
        m = jnp.maximum(jnp.maximum(ss[0], ss[1]), ss[2])
        es = [jnp.exp(s_r - m) for s_r in ss]
        den = es[0] + es[1] + es[2]
        o = (es[0] * vs[0] + es[1] * vs[1] + es[2] * vs[2]) / den
        out_ref[...] = (
            lax.dot_general(o, wp[...], dt, preferred_element_type=jnp.float32)
            + bp_ref[...] + f)

    def gview(r):
        return pl.BlockSpec((BN, D), lambda i, r=r: (nb * r + i, 0))

    def cview(r):
        return pl.BlockSpec((BN, _L), lambda i, r=r: (nb * r + i, 0))

    wspec = pl.BlockSpec((D, D), lambda i: (0, 0))
    bspec = pl.BlockSpec((1, D), lambda i: (0, 0))
    return pl.pallas_call(
        body,
        grid=(grid,),
        in_specs=[
            pl.BlockSpec((BN, D), lambda i: (i, 0)),
            gview(0), gview(1), gview(2),
            cview(0), cview(1), cview(2),
            pl.BlockSpec((R, D), lambda i: (0, 0)),
            wspec, wspec, wspec, wspec, bspec, wspec, bspec,
        ],
        out_specs=pl.BlockSpec((BN, D), lambda i: (i, 0)),
        out_shape=jax.ShapeDtypeStruct((N, D), jnp.float32),
    )(feat, G, G, G, C, C, C, emb, Wq, Wk, Wv, Wa, ba2, Wp, bp2)


def kernel(feat, edge_index, edge_type, emb, Wq, Wk, Wv, Wa, ba, Wp, bp):
    N, D = feat.shape
    R = emb.shape[0]
    src = edge_index[0]
    dst = edge_index[1]
    seg = edge_type * N + dst
    featrows = feat.reshape(N // 8, 8, 2, 128).transpose(0, 2, 1, 3)
    featrows = featrows.reshape(2 * N, 128)
    G, C = _sc_segsum(featrows, src, seg, R * N)
    return _tc_attn(feat, G, C, emb, Wq, Wk, Wv, Wa,
                    ba.reshape(1, -1), Wp, bp.reshape(1, -1))
